# baseline (speedup 1.0000x reference)
import functools

import jax
import jax.numpy as jnp
from jax import lax
from jax.experimental import pallas as pl
from jax.experimental.pallas import tpu as pltpu
from jax.experimental.pallas import tpu_sc as plsc

D = 256
CHUNK = 128
NBUF = 3


@functools.cache
def _build_gather(nb, nt):
    info = plsc.get_sparse_core_info()
    n_workers = info.num_cores * info.num_subcores
    per_w = (nb * nt) // n_workers
    w_per_row = nt // per_w
    n_chunks = per_w // CHUNK
    mesh = plsc.VectorSubcoreMesh(core_axis_name="c", subcore_axis_name="s")

    @functools.partial(
        pl.kernel,
        mesh=mesh,
        out_type=jax.ShapeDtypeStruct((nb, nt, D), jnp.float32),
        scratch_types=[
            pltpu.VMEM((per_w,), jnp.int32),
            pltpu.VMEM((NBUF, CHUNK, D), jnp.float32),
            pltpu.SemaphoreType.DMA((NBUF,)),
            pltpu.SemaphoreType.DMA((NBUF,)),
        ],
    )
    def gather_kernel(table_hbm, idx_hbm, out_hbm, idx_v, rows_v, gsem, wsem):
        wid = lax.axis_index("s") * info.num_cores + lax.axis_index("c")

        def flat_base(i):
            return i * (n_workers * CHUNK) + wid * CHUNK

        for i in range(n_chunks):
            fb = flat_base(i)
            pltpu.sync_copy(
                idx_hbm.at[fb // nt, pl.ds(fb % nt, CHUNK)],
                idx_v.at[pl.ds(i * CHUNK, CHUNK)])

        def gather(i):
            return pltpu.async_copy(
                table_hbm.at[idx_v.at[pl.ds(i * CHUNK, CHUNK)]],
                rows_v.at[i % NBUF], gsem.at[i % NBUF])

        def write(i):
            fb = flat_base(i)
            return pltpu.async_copy(
                rows_v.at[i % NBUF],
                out_hbm.at[fb // nt, pl.ds(fb % nt, CHUNK)],
                wsem.at[i % NBUF])

        gathers = [None] * n_chunks
        writes = [None] * n_chunks
        for i in range(min(NBUF - 1, n_chunks)):
            gathers[i] = gather(i)
        for i in range(n_chunks):
            gathers[i].wait()
            writes[i] = write(i)
            nxt = i + NBUF - 1
            if nxt < n_chunks:
                if i >= 1:
                    writes[i - 1].wait()
                gathers[nxt] = gather(nxt)
        for i in range(max(0, n_chunks - NBUF), n_chunks):
            writes[i].wait()

    return gather_kernel


def kernel(x, x_embed_weight, batch_size=1):
    nb, nt = x.shape
    return _build_gather(nb, nt)(x_embed_weight, x)


# speedup vs baseline: 1.0806x; 1.0806x over previous
import functools

import jax
import jax.numpy as jnp
from jax import lax
from jax.experimental import pallas as pl
from jax.experimental.pallas import tpu as pltpu
from jax.experimental.pallas import tpu_sc as plsc

D = 256
CHUNK = 64
NBUF = 7


@functools.cache
def _build_gather(nb, nt):
    info = plsc.get_sparse_core_info()
    n_workers = info.num_cores * info.num_subcores
    per_w = (nb * nt) // n_workers
    w_per_row = nt // per_w
    n_chunks = per_w // CHUNK
    mesh = plsc.VectorSubcoreMesh(core_axis_name="c", subcore_axis_name="s")

    @functools.partial(
        pl.kernel,
        mesh=mesh,
        out_type=jax.ShapeDtypeStruct((nb, nt, D), jnp.float32),
        scratch_types=[
            pltpu.VMEM((per_w,), jnp.int32),
            pltpu.VMEM((NBUF, CHUNK, D), jnp.float32),
            pltpu.SemaphoreType.DMA((NBUF,)),
            pltpu.SemaphoreType.DMA((NBUF,)),
        ],
    )
    def gather_kernel(table_hbm, idx_hbm, out_hbm, idx_v, rows_v, gsem, wsem):
        wid = lax.axis_index("s") * info.num_cores + lax.axis_index("c")
        row = wid // w_per_row
        col = (wid % w_per_row) * per_w
        pltpu.sync_copy(idx_hbm.at[row, pl.ds(col, per_w)], idx_v)

        def gather(i):
            return pltpu.async_copy(
                table_hbm.at[idx_v.at[pl.ds(i * CHUNK, CHUNK)]],
                rows_v.at[i % NBUF], gsem.at[i % NBUF])

        def write(i):
            return pltpu.async_copy(
                rows_v.at[i % NBUF],
                out_hbm.at[row, pl.ds(col + i * CHUNK, CHUNK)],
                wsem.at[i % NBUF])

        gathers = [None] * n_chunks
        writes = [None] * n_chunks
        for i in range(min(NBUF - 1, n_chunks)):
            gathers[i] = gather(i)
        for i in range(n_chunks):
            gathers[i].wait()
            writes[i] = write(i)
            nxt = i + NBUF - 1
            if nxt < n_chunks:
                if i >= 1:
                    writes[i - 1].wait()
                gathers[nxt] = gather(nxt)
        for i in range(max(0, n_chunks - NBUF), n_chunks):
            writes[i].wait()

    return gather_kernel


def kernel(x, x_embed_weight, batch_size=1):
    nb, nt = x.shape
    return _build_gather(nb, nt)(x_embed_weight, x)
